# baseline (speedup 1.0000x reference)
import functools

import jax
import jax.numpy as jnp
from jax import lax
from jax.experimental import pallas as pl
from jax.experimental.pallas import tpu as pltpu
from jax.experimental.pallas import tpu_sc as plsc

D = 128
H = 64
C = 128
M = 16
NW = 32
LANES = 16

HALVES = {"bond": (0, 1), "angle": (0, 0, 1), "torsion": (0, 0, 1, 1)}


def _proj_body(x_ref, w_ref, t_ref):
    t_ref[...] = jnp.dot(x_ref[...], w_ref[0],
                         preferred_element_type=jnp.float32)


def _project(x, wstack):
    n_atoms = x.shape[0]
    blk = 1000
    nblk = n_atoms // blk
    npiece = wstack.shape[0]
    return pl.pallas_call(
        _proj_body,
        grid=(nblk, npiece),
        in_specs=[
            pl.BlockSpec((blk, D), lambda i, h: (i, 0)),
            pl.BlockSpec((1, D, D), lambda i, h: (h, 0, 0)),
        ],
        out_specs=pl.BlockSpec((blk, D), lambda i, h: (h * nblk + i, 0)),
        out_shape=jax.ShapeDtypeStruct((npiece * n_atoms, D), jnp.float32),
    )(x, wstack)


def _sc_gather_body(table, bidx, aidx, tidx, zb, za, zt, ibuf, gbuf, zbuf,
                    sem):
    cid = lax.axis_index("c")
    sid = lax.axis_index("s")
    wid = sid * 2 + cid

    def do_type(idx_hbm, z_hbm, n_chunks, halves):
        k = len(halves)
        start = (wid * n_chunks) // NW
        end = ((wid + 1) * n_chunks) // NW

        def chunk(g, carry):
            pltpu.sync_copy(idx_hbm.at[pl.ds(g * (C * k), C * k)],
                            ibuf.at[pl.ds(0, C * k)])
            waits = []
            for j in range(k):
                waits.append(pltpu.async_copy(
                    table.at[ibuf.at[pl.ds(j * C, C)]],
                    gbuf.at[pl.ds(j * C, C)], sem))
            for w in waits:
                w.wait()

            def acc_row(r, carry2):
                e = r * k
                for j in range(H // LANES):
                    f = None
                    rv = None
                    for p, hf in enumerate(halves):
                        gf = gbuf[e + p, pl.ds(hf * H + j * LANES, LANES)]
                        gr = gbuf[e + p,
                                  pl.ds((1 - hf) * H + j * LANES, LANES)]
                        f = gf if f is None else f + gf
                        rv = gr if rv is None else rv + gr
                    zbuf[r, pl.ds(j * LANES, LANES)] = f
                    zbuf[r, pl.ds(H + j * LANES, LANES)] = rv
                return carry2

            lax.fori_loop(0, C, acc_row, 0)
            pltpu.sync_copy(zbuf, z_hbm.at[g])
            return carry

        lax.fori_loop(start, end, chunk, 0)

    do_type(bidx, zb, zb.shape[0], HALVES["bond"])
    do_type(aidx, za, za.shape[0], HALVES["angle"])
    do_type(tidx, zt, zt.shape[0], HALVES["torsion"])


def _sc_gather(table, bidx, aidx, tidx, ncb, nca, nct):
    mesh = plsc.VectorSubcoreMesh(core_axis_name="c", subcore_axis_name="s")
    out_type = [
        jax.ShapeDtypeStruct((ncb, C, 2 * H), jnp.float32),
        jax.ShapeDtypeStruct((nca, C, 2 * H), jnp.float32),
        jax.ShapeDtypeStruct((nct, C, 2 * H), jnp.float32),
    ]
    scratch = [
        pltpu.VMEM((4 * C,), jnp.int32),
        pltpu.VMEM((4 * C, 128), jnp.float32),
        pltpu.VMEM((C, 2 * H), jnp.float32),
        pltpu.SemaphoreType.DMA,
    ]
    fn = pl.kernel(_sc_gather_body, out_type=out_type, mesh=mesh,
                   scratch_types=scratch,
                   compiler_params=pltpu.CompilerParams(
                       use_tc_tiling_on_sc=True))
    return fn(table, bidx, aidx, tidx)


def _tail_body(z_ref, r_ref, wr_ref, b1_ref, w2_ref, b2_ref, w3_ref, b3_ref,
               o_ref):
    z = z_ref[...].reshape(M * C, 2 * H)
    zf = z[:, 0:H]
    zr = z[:, H:2 * H]
    base = r_ref[...] * wr_ref[...] + b1_ref[...]
    h1f = jnp.maximum(zf + base, 0.0)
    h1r = jnp.maximum(zr + base, 0.0)
    w2 = w2_ref[...]
    h2f = jnp.maximum(
        jnp.dot(h1f, w2, preferred_element_type=jnp.float32) + b2_ref[...], 0.0)
    h2r = jnp.maximum(
        jnp.dot(h1r, w2, preferred_element_type=jnp.float32) + b2_ref[...], 0.0)
    o_ref[...] = (jnp.dot(h2f + h2r, w3_ref[...],
                          preferred_element_type=jnp.float32) + b3_ref[...])


def _tail(z4, repr_, wr, b1, w2, b2, w3, b3):
    n = repr_.shape[0]
    nch = z4.shape[0]
    grid = (nch + M - 1) // M
    n_out = w3.shape[1]
    return pl.pallas_call(
        _tail_body,
        grid=(grid,),
        in_specs=[
            pl.BlockSpec((M, C, 2 * H), lambda i: (i, 0, 0)),
            pl.BlockSpec((M * C, 1), lambda i: (i, 0)),
            pl.BlockSpec((1, H), lambda i: (0, 0)),
            pl.BlockSpec((1, H), lambda i: (0, 0)),
            pl.BlockSpec((H, H), lambda i: (0, 0)),
            pl.BlockSpec((1, H), lambda i: (0, 0)),
            pl.BlockSpec((H, n_out), lambda i: (0, 0)),
            pl.BlockSpec((1, n_out), lambda i: (0, 0)),
        ],
        out_specs=pl.BlockSpec((M * C, n_out), lambda i: (i, 0)),
        out_shape=jax.ShapeDtypeStruct((n, n_out), jnp.float32),
    )(z4, repr_, wr, b1, w2, b2, w3, b3)


def _prep_idx(idx, offsets, npad):
    n, k = idx.shape
    flat = (idx.astype(jnp.int32)
            + jnp.asarray(offsets, jnp.int32)[None, :]).reshape(-1)
    return jnp.pad(flat, (0, k * npad - k * n))


def kernel(x, bond_idx, angle_idx, torsion_idx, bond_repr, angle_repr,
           torsion_repr, bond_params, angle_params, torsion_params):
    wb1 = bond_params[0]
    wa1 = angle_params[0]
    wt1 = torsion_params[0]
    na_ = x.shape[0]

    wstack = jnp.stack([
        jnp.concatenate([wb1[0:D], wb1[D:2 * D]], axis=1),
        jnp.concatenate([wa1[0:D], wa1[2 * D:3 * D]], axis=1),
        jnp.concatenate([wa1[D:2 * D], wa1[D:2 * D]], axis=1),
        jnp.concatenate([wt1[0:D], wt1[3 * D:4 * D]], axis=1),
        jnp.concatenate([wt1[D:2 * D], wt1[2 * D:3 * D]], axis=1),
    ])

    table = _project(x, wstack)

    nb, naf, nt = bond_idx.shape[0], angle_idx.shape[0], torsion_idx.shape[0]
    step = C * M
    ncb = ((nb + step - 1) // step) * M
    nca = ((naf + step - 1) // step) * M
    nct = ((nt + step - 1) // step) * M

    bidx = _prep_idx(bond_idx, [0, 0], ncb * C)
    aidx = _prep_idx(angle_idx, [na_, 2 * na_, na_], nca * C)
    tidx = _prep_idx(torsion_idx, [3 * na_, 4 * na_, 4 * na_, 3 * na_],
                     nct * C)

    zb, za, zt = _sc_gather(table, bidx, aidx, tidx, ncb, nca, nct)

    def tail_for(z4, repr_, params):
        w1, b1, w2, b2, w3, b3 = params
        wr = w1[-1:, :]
        return _tail(z4, repr_, wr, b1.reshape(1, H), w2, b2.reshape(1, H),
                     w3, (2.0 * b3).reshape(1, -1))

    ob = tail_for(zb, bond_repr, bond_params)
    oa = tail_for(za, angle_repr, angle_params)
    ot = tail_for(zt, torsion_repr, torsion_params)

    return jnp.concatenate([ob, oa, ot], axis=0)
